# Initial kernel scaffold; baseline (speedup 1.0000x reference)
import jax
import jax.numpy as jnp
from jax.experimental import pallas as pl


def kernel(X, emb_w, def_w, w_w):
    raise NotImplementedError("write your pallas kernel here")


# speedup vs baseline: 25.1706x; 25.1706x over previous
import jax
import jax.numpy as jnp
from jax import lax
from jax.experimental import pallas as pl
from jax.experimental.pallas import tpu as pltpu
from jax.experimental.pallas import tpu_sc as plsc

B, F, V, D, NSEEN = 16384, 26, 1000, 64, 20
FN = F * NSEEN
NP = F // 2
BP = B * NP
NROW = NP * NSEEN * NSEEN
NW = 32
ROWS_PER_W = BP // NW
BLK = 256
NBLK = ROWS_PER_W // BLK
NSTREAM = BLK // 128


def _fuse_body(emb_ref, w_ref, def_ref, t_ref):
    w = w_ref[...]
    t_ref[...] = w * emb_ref[...] + (1.0 - w) * def_ref[...]


def _gather_body(xe_hbm, xo_hbm, tp_hbm, out_hbm,
                 xe_ref, xo_ref, i0, i1, rows_ref, sem):
    wid = lax.axis_index("s") * 2 + lax.axis_index("c")
    lane = lax.broadcasted_iota(jnp.int32, (16,), 0)
    idx_refs = [i0, i1]

    def block(blk, carry):
        base = wid * ROWS_PER_W + blk * BLK
        pltpu.sync_copy(xe_hbm.at[pl.ds(base, BLK)], xe_ref)
        pltpu.sync_copy(xo_hbm.at[pl.ds(base, BLK)], xo_ref)
        for j in range(BLK // 16):
            xe = xe_ref[pl.ds(j * 16, 16)]
            xo = xo_ref[pl.ds(j * 16, 16)]
            p = lax.rem(base + j * 16 + lane, NP)
            idx = p * (NSEEN * NSEEN) + xe * NSEEN + xo
            idx_refs[j // 8][pl.ds((j % 8) * 16, 16)] = idx
        copies = [
            pltpu.async_copy(tp_hbm.at[idx_refs[s]],
                             rows_ref.at[pl.ds(s * 128, 128)], sem)
            for s in range(NSTREAM)
        ]
        for c in copies:
            c.wait()
        pltpu.sync_copy(rows_ref, out_hbm.at[pl.ds(base, BLK)])
        return carry

    lax.fori_loop(0, NBLK, block, 0)


def kernel(X, emb_w, def_w, w_w):
    emb_e = emb_w[:, :NSEEN, :].reshape(FN, D)
    w_e = w_w[:, :NSEEN, :].reshape(FN, 1)
    def_e = jnp.broadcast_to(def_w[:, None, :], (F, NSEEN, D)).reshape(FN, D)
    t = pl.pallas_call(
        _fuse_body,
        out_shape=jax.ShapeDtypeStruct((FN, D), jnp.float32),
    )(emb_e, w_e, def_e)

    t3 = t.reshape(NP, 2, NSEEN, D)
    te = jnp.broadcast_to(t3[:, 0, :, None, :], (NP, NSEEN, NSEEN, D))
    to = jnp.broadcast_to(t3[:, 1, None, :, :], (NP, NSEEN, NSEEN, D))
    tp = jnp.concatenate([te, to], axis=-1).reshape(NROW, 2 * D)

    xe_flat = X[:, 0::2].reshape(BP)
    xo_flat = X[:, 1::2].reshape(BP)

    mesh = plsc.VectorSubcoreMesh(core_axis_name="c", subcore_axis_name="s")
    gather = pl.kernel(
        _gather_body,
        mesh=mesh,
        out_type=jax.ShapeDtypeStruct((BP, 2 * D), jnp.float32),
        scratch_types=[
            pltpu.VMEM((BLK,), jnp.int32),
            pltpu.VMEM((BLK,), jnp.int32),
            pltpu.VMEM((128,), jnp.int32),
            pltpu.VMEM((128,), jnp.int32),
            pltpu.VMEM((BLK, 2 * D), jnp.float32),
            pltpu.SemaphoreType.DMA,
        ],
    )
    out_pairs = gather(xe_flat, xo_flat, tp)
    return out_pairs.reshape(B, F, D)


# speedup vs baseline: 26.5876x; 1.0563x over previous
import jax
import jax.numpy as jnp
from jax import lax
from jax.experimental import pallas as pl
from jax.experimental.pallas import tpu as pltpu
from jax.experimental.pallas import tpu_sc as plsc

B, F, V, D, NSEEN = 16384, 26, 1000, 64, 20
FN = F * NSEEN
NP = F // 2
BP = B * NP
NROW = NP * NSEEN * NSEEN
NW = 32
ROWS_PER_W = BP // NW
BLK = 256
NBLK = ROWS_PER_W // BLK
NSTREAM = BLK // 128


def _fuse_body(emb_ref, w_ref, def_ref, t_ref):
    w = w_ref[...]
    t_ref[...] = w * emb_ref[...] + (1.0 - w) * def_ref[...]


def _gather_body(xe_hbm, xo_hbm, tp_hbm, out_hbm,
                 xe0, xe1, xo0, xo1, i00, i01, i10, i11, r0, r1,
                 gsem0, gsem1, wsem0, wsem1):
    wid = lax.axis_index("s") * 2 + lax.axis_index("c")
    lane = lax.broadcasted_iota(jnp.int32, (16,), 0)
    slots = [
        (xe0, xo0, (i00, i01), r0, gsem0, wsem0),
        (xe1, xo1, (i10, i11), r1, gsem1, wsem1),
    ]

    def prep(g, slot):
        xe_b, xo_b, ibs, rows_b, gsem, _ = slot
        base = wid * ROWS_PER_W + g * BLK
        pltpu.sync_copy(xe_hbm.at[pl.ds(base, BLK)], xe_b)
        pltpu.sync_copy(xo_hbm.at[pl.ds(base, BLK)], xo_b)
        for j in range(BLK // 16):
            xe = xe_b[pl.ds(j * 16, 16)]
            xo = xo_b[pl.ds(j * 16, 16)]
            p = lax.rem(base + j * 16 + lane, NP)
            ibs[j // 8][pl.ds((j % 8) * 16, 16)] = (
                p * (NSEEN * NSEEN) + xe * NSEEN + xo)
        return [
            pltpu.async_copy(tp_hbm.at[ibs[s]],
                             rows_b.at[pl.ds(s * 128, 128)], gsem)
            for s in range(NSTREAM)
        ]

    pend_g = {0: prep(0, slots[0]), 1: None}
    pend_w = {0: None, 1: None}
    for g in range(NBLK):
        s = g % 2
        s2 = (g + 1) % 2
        if g + 1 < NBLK:
            if pend_w[s2] is not None:
                pend_w[s2].wait()
            pend_g[s2] = prep(g + 1, slots[s2])
        for c in pend_g[s]:
            c.wait()
        base = wid * ROWS_PER_W + g * BLK
        pend_w[s] = pltpu.async_copy(
            slots[s][3], out_hbm.at[pl.ds(base, BLK)], slots[s][5])
    pend_w[0].wait()
    pend_w[1].wait()


def kernel(X, emb_w, def_w, w_w):
    emb_e = emb_w[:, :NSEEN, :].reshape(FN, D)
    w_e = w_w[:, :NSEEN, :].reshape(FN, 1)
    def_e = jnp.broadcast_to(def_w[:, None, :], (F, NSEEN, D)).reshape(FN, D)
    t = pl.pallas_call(
        _fuse_body,
        out_shape=jax.ShapeDtypeStruct((FN, D), jnp.float32),
    )(emb_e, w_e, def_e)

    t3 = t.reshape(NP, 2, NSEEN, D)
    te = jnp.broadcast_to(t3[:, 0, :, None, :], (NP, NSEEN, NSEEN, D))
    to = jnp.broadcast_to(t3[:, 1, None, :, :], (NP, NSEEN, NSEEN, D))
    tp = jnp.concatenate([te, to], axis=-1).reshape(NROW, 2 * D)

    xe_flat = X[:, 0::2].reshape(BP)
    xo_flat = X[:, 1::2].reshape(BP)

    mesh = plsc.VectorSubcoreMesh(core_axis_name="c", subcore_axis_name="s")
    gather = pl.kernel(
        _gather_body,
        mesh=mesh,
        out_type=jax.ShapeDtypeStruct((BP, 2 * D), jnp.float32),
        scratch_types=[
            pltpu.VMEM((BLK,), jnp.int32),
            pltpu.VMEM((BLK,), jnp.int32),
            pltpu.VMEM((BLK,), jnp.int32),
            pltpu.VMEM((BLK,), jnp.int32),
            pltpu.VMEM((128,), jnp.int32),
            pltpu.VMEM((128,), jnp.int32),
            pltpu.VMEM((128,), jnp.int32),
            pltpu.VMEM((128,), jnp.int32),
            pltpu.VMEM((BLK, 2 * D), jnp.float32),
            pltpu.VMEM((BLK, 2 * D), jnp.float32),
            pltpu.SemaphoreType.DMA,
            pltpu.SemaphoreType.DMA,
            pltpu.SemaphoreType.DMA,
            pltpu.SemaphoreType.DMA,
        ],
    )
    out_pairs = gather(xe_flat, xo_flat, tp)
    return out_pairs.reshape(B, F, D)


# speedup vs baseline: 31.3129x; 1.1777x over previous
import jax
import jax.numpy as jnp
from jax import lax
from jax.experimental import pallas as pl
from jax.experimental.pallas import tpu as pltpu
from jax.experimental.pallas import tpu_sc as plsc

B, F, V, D, NSEEN = 16384, 26, 1000, 64, 20
FN = F * NSEEN
NP = F // 2
BP = B * NP
NROW = NP * NSEEN * NSEEN
NW = 32
ROWS_PER_W = BP // NW
BLK = 256
NBLK = ROWS_PER_W // BLK
NSTREAM = BLK // 128


def _fuse_body(emb_ref, w_ref, def_ref, t_ref):
    w = w_ref[...]
    t_ref[...] = w * emb_ref[...] + (1.0 - w) * def_ref[...]


CB = 512


def _tr_body(p_ref, o_ref):
    x = p_ref[...]
    for p in range(NP):
        o_ref[p, :, :] = x[:, p, :].T


def _gather_body(xe_hbm, xo_hbm, tp_hbm, out_hbm,
                 xe0, xe1, xo0, xo1, i00, i01, i10, i11, r0, r1,
                 gsem0, gsem1, wsem0, wsem1):
    wid = lax.axis_index("s") * 2 + lax.axis_index("c")
    lane = lax.broadcasted_iota(jnp.int32, (16,), 0)
    slots = [
        (xe0, xo0, (i00, i01), r0, gsem0, wsem0),
        (xe1, xo1, (i10, i11), r1, gsem1, wsem1),
    ]

    def prep(g, slot):
        xe_b, xo_b, ibs, rows_b, gsem, _ = slot
        base = wid * ROWS_PER_W + g * BLK
        pltpu.sync_copy(xe_hbm.at[pl.ds(base, BLK)], xe_b)
        pltpu.sync_copy(xo_hbm.at[pl.ds(base, BLK)], xo_b)
        for j in range(BLK // 16):
            xe = xe_b[pl.ds(j * 16, 16)]
            xo = xo_b[pl.ds(j * 16, 16)]
            p = lax.rem(base + j * 16 + lane, NP)
            ibs[j // 8][pl.ds((j % 8) * 16, 16)] = (
                p * (NSEEN * NSEEN) + xe * NSEEN + xo)
        return [
            pltpu.async_copy(tp_hbm.at[ibs[s]],
                             rows_b.at[pl.ds(s * 128, 128)], gsem)
            for s in range(NSTREAM)
        ]

    pend_g = {0: prep(0, slots[0]), 1: None}
    pend_w = {0: None, 1: None}
    for g in range(NBLK):
        s = g % 2
        s2 = (g + 1) % 2
        if g + 1 < NBLK:
            if pend_w[s2] is not None:
                pend_w[s2].wait()
            pend_g[s2] = prep(g + 1, slots[s2])
        for c in pend_g[s]:
            c.wait()
        base = wid * ROWS_PER_W + g * BLK
        pend_w[s] = pltpu.async_copy(
            slots[s][3], out_hbm.at[pl.ds(base, BLK)], slots[s][5])
    pend_w[0].wait()
    pend_w[1].wait()


def kernel(X, emb_w, def_w, w_w):
    emb_e = emb_w[:, :NSEEN, :].reshape(FN, D)
    w_e = w_w[:, :NSEEN, :].reshape(FN, 1)
    def_e = jnp.broadcast_to(def_w[:, None, :], (F, NSEEN, D)).reshape(FN, D)
    t = pl.pallas_call(
        _fuse_body,
        out_shape=jax.ShapeDtypeStruct((FN, D), jnp.float32),
    )(emb_e, w_e, def_e)

    t3 = t.reshape(NP, 2, NSEEN, D)
    te = jnp.broadcast_to(t3[:, 0, :, None, :], (NP, NSEEN, NSEEN, D))
    to = jnp.broadcast_to(t3[:, 1, None, :, :], (NP, NSEEN, NSEEN, D))
    tp = jnp.concatenate([te, to], axis=-1).reshape(NROW, 2 * D)

    xe_flat = X[:, 0::2].reshape(BP)
    xo_flat = X[:, 1::2].reshape(BP)

    mesh = plsc.VectorSubcoreMesh(core_axis_name="c", subcore_axis_name="s")
    gather = pl.kernel(
        _gather_body,
        mesh=mesh,
        out_type=jax.ShapeDtypeStruct((BP, 2 * D), jnp.float32),
        scratch_types=[
            pltpu.VMEM((BLK,), jnp.int32),
            pltpu.VMEM((BLK,), jnp.int32),
            pltpu.VMEM((BLK,), jnp.int32),
            pltpu.VMEM((BLK,), jnp.int32),
            pltpu.VMEM((128,), jnp.int32),
            pltpu.VMEM((128,), jnp.int32),
            pltpu.VMEM((128,), jnp.int32),
            pltpu.VMEM((128,), jnp.int32),
            pltpu.VMEM((BLK, 2 * D), jnp.float32),
            pltpu.VMEM((BLK, 2 * D), jnp.float32),
            pltpu.SemaphoreType.DMA,
            pltpu.SemaphoreType.DMA,
            pltpu.SemaphoreType.DMA,
            pltpu.SemaphoreType.DMA,
        ],
    )
    out_pairs = gather(xe_flat, xo_flat, tp)

    out_t = pl.pallas_call(
        _tr_body,
        grid=(B // CB,),
        in_specs=[pl.BlockSpec((CB, NP, 2 * D), lambda g: (g, 0, 0))],
        out_specs=pl.BlockSpec((NP, 2 * D, CB), lambda g: (0, 0, g)),
        out_shape=jax.ShapeDtypeStruct((NP, 2 * D, B), jnp.float32),
    )(out_pairs.reshape(B, NP, 2 * D))
    return jnp.transpose(out_t.reshape(F, D, B), (2, 0, 1))


# speedup vs baseline: 32.6662x; 1.0432x over previous
import jax
import jax.numpy as jnp
from jax import lax
from jax.experimental import pallas as pl
from jax.experimental.pallas import tpu as pltpu
from jax.experimental.pallas import tpu_sc as plsc

B, F, V, D, NSEEN = 16384, 26, 1000, 64, 20
FN = F * NSEEN
NP = F // 2
BP = B * NP
NROW = NP * NSEEN * NSEEN
NW = 32
BLK = 256
NSTREAM = BLK // 128

NCHUNK = 2
CBATCH = B // NCHUNK
CBP = CBATCH * NP
ROWS_W = CBP // NW
NBLK = ROWS_W // BLK
CB = 512


def _fuse_body(emb_ref, w_ref, def_ref, t_ref):
    w = w_ref[...]
    t_ref[...] = w * emb_ref[...] + (1.0 - w) * def_ref[...]


def _tr_body(p_ref, o_ref):
    x = p_ref[...]
    for p in range(NP):
        o_ref[p, :, :] = x[:, p, :].T


def _tr_body_alias(buf_ref, p_ref, o_ref):
    del buf_ref
    x = p_ref[...]
    for p in range(NP):
        o_ref[p, :, :] = x[:, p, :].T


def _make_gather_body(chunk):
    c0 = chunk * CBP

    def _gather_body(xe_hbm, xo_hbm, tp_hbm, out_hbm,
                     xe0, xe1, xo0, xo1, i00, i01, i10, i11, r0, r1,
                     gsem0, gsem1, wsem0, wsem1):
        wid = lax.axis_index("s") * 2 + lax.axis_index("c")
        lane = lax.broadcasted_iota(jnp.int32, (16,), 0)
        slots = [
            (xe0, xo0, (i00, i01), r0, gsem0, wsem0),
            (xe1, xo1, (i10, i11), r1, gsem1, wsem1),
        ]

        def prep(g, slot):
            xe_b, xo_b, ibs, rows_b, gsem, _ = slot
            base = c0 + wid * ROWS_W + g * BLK
            pltpu.sync_copy(xe_hbm.at[pl.ds(base, BLK)], xe_b)
            pltpu.sync_copy(xo_hbm.at[pl.ds(base, BLK)], xo_b)
            for j in range(BLK // 16):
                xe = xe_b[pl.ds(j * 16, 16)]
                xo = xo_b[pl.ds(j * 16, 16)]
                p = lax.rem(base + j * 16 + lane, NP)
                ibs[j // 8][pl.ds((j % 8) * 16, 16)] = (
                    p * (NSEEN * NSEEN) + xe * NSEEN + xo)
            return [
                pltpu.async_copy(tp_hbm.at[ibs[s]],
                                 rows_b.at[pl.ds(s * 128, 128)], gsem)
                for s in range(NSTREAM)
            ]

        pend_g = {0: prep(0, slots[0]), 1: None}
        pend_w = {0: None, 1: None}
        for g in range(NBLK):
            s = g % 2
            s2 = (g + 1) % 2
            if g + 1 < NBLK:
                if pend_w[s2] is not None:
                    pend_w[s2].wait()
                pend_g[s2] = prep(g + 1, slots[s2])
            for c in pend_g[s]:
                c.wait()
            loc = wid * ROWS_W + g * BLK
            pend_w[s] = pltpu.async_copy(
                slots[s][3], out_hbm.at[pl.ds(loc, BLK)], slots[s][5])
        pend_w[(NBLK - 1) % 2].wait()
        pend_w[NBLK % 2].wait()

    return _gather_body


def kernel(X, emb_w, def_w, w_w):
    emb_e = emb_w[:, :NSEEN, :].reshape(FN, D)
    w_e = w_w[:, :NSEEN, :].reshape(FN, 1)
    def_e = jnp.broadcast_to(def_w[:, None, :], (F, NSEEN, D)).reshape(FN, D)
    t = pl.pallas_call(
        _fuse_body,
        out_shape=jax.ShapeDtypeStruct((FN, D), jnp.float32),
    )(emb_e, w_e, def_e)

    t3 = t.reshape(NP, 2, NSEEN, D)
    te = jnp.broadcast_to(t3[:, 0, :, None, :], (NP, NSEEN, NSEEN, D))
    to = jnp.broadcast_to(t3[:, 1, None, :, :], (NP, NSEEN, NSEEN, D))
    tp = jnp.concatenate([te, to], axis=-1).reshape(NROW, 2 * D)

    xe_flat = X[:, 0::2].reshape(BP)
    xo_flat = X[:, 1::2].reshape(BP)

    mesh = plsc.VectorSubcoreMesh(core_axis_name="c", subcore_axis_name="s")
    scratch = [
        pltpu.VMEM((BLK,), jnp.int32),
        pltpu.VMEM((BLK,), jnp.int32),
        pltpu.VMEM((BLK,), jnp.int32),
        pltpu.VMEM((BLK,), jnp.int32),
        pltpu.VMEM((128,), jnp.int32),
        pltpu.VMEM((128,), jnp.int32),
        pltpu.VMEM((128,), jnp.int32),
        pltpu.VMEM((128,), jnp.int32),
        pltpu.VMEM((BLK, 2 * D), jnp.float32),
        pltpu.VMEM((BLK, 2 * D), jnp.float32),
        pltpu.SemaphoreType.DMA,
        pltpu.SemaphoreType.DMA,
        pltpu.SemaphoreType.DMA,
        pltpu.SemaphoreType.DMA,
    ]

    nblk_tr = CBATCH // CB
    buf = None
    for c in range(NCHUNK):
        gather = pl.kernel(
            _make_gather_body(c),
            mesh=mesh,
            out_type=jax.ShapeDtypeStruct((CBP, 2 * D), jnp.float32),
            scratch_types=scratch,
        )
        pc = gather(xe_flat, xo_flat, tp).reshape(CBATCH, NP, 2 * D)
        if buf is None:
            buf = pl.pallas_call(
                _tr_body,
                grid=(nblk_tr,),
                in_specs=[pl.BlockSpec((CB, NP, 2 * D), lambda g: (g, 0, 0))],
                out_specs=pl.BlockSpec((NP, 2 * D, CB), lambda g: (0, 0, g)),
                out_shape=jax.ShapeDtypeStruct((NP, 2 * D, B), jnp.float32),
            )(pc)
        else:
            buf = pl.pallas_call(
                _tr_body_alias,
                grid=(nblk_tr,),
                in_specs=[
                    pl.BlockSpec(memory_space=pltpu.MemorySpace.HBM),
                    pl.BlockSpec((CB, NP, 2 * D), lambda g: (g, 0, 0)),
                ],
                out_specs=pl.BlockSpec(
                    (NP, 2 * D, CB),
                    lambda g, cc=c: (0, 0, cc * nblk_tr + g)),
                out_shape=jax.ShapeDtypeStruct((NP, 2 * D, B), jnp.float32),
                input_output_aliases={0: 0},
            )(buf, pc)

    return jnp.transpose(buf.reshape(F, D, B), (2, 0, 1))


# speedup vs baseline: 35.2126x; 1.0780x over previous
import jax
import jax.numpy as jnp
from jax import lax
from jax.experimental import pallas as pl
from jax.experimental.pallas import tpu as pltpu
from jax.experimental.pallas import tpu_sc as plsc

B, F, V, D, NSEEN = 16384, 26, 1000, 64, 20
FN = F * NSEEN
NP = F // 2
BP = B * NP
NROW = NP * NSEEN * NSEEN
NROWP = 5248
NW = 32
BLK = 256
NSTREAM = BLK // 128

NCHUNK = 2
CBATCH = B // NCHUNK
CBP = CBATCH * NP
ROWS_W = CBP // NW
NBLK = ROWS_W // BLK
CB = 512


def _fuse_body(emb_ref, w_ref, def_ref, t_ref):
    w = w_ref[...]
    t_ref[...] = w * emb_ref[...] + (1.0 - w) * def_ref[...]


def _tr_body(p_ref, o_ref):
    x = p_ref[...]
    for p in range(NP):
        o_ref[p, :, :] = x[:, p, :].T


def _tr_body_alias(buf_ref, p_ref, o_ref):
    del buf_ref
    x = p_ref[...]
    for p in range(NP):
        o_ref[p, :, :] = x[:, p, :].T


def _make_gather_body(chunk):
    c0 = chunk * CBP

    def _gather_body(xe_hbm, xo_hbm, tp_hbm, out_hbm,
                     xe0, xe1, xo0, xo1, i00, i01, i10, i11, r0, r1, tp_sp,
                     gsem0, gsem1, wsem0, wsem1):
        wid = lax.axis_index("s") * 2 + lax.axis_index("c")
        sid = lax.axis_index("s")
        lane = lax.broadcasted_iota(jnp.int32, (16,), 0)
        slots = [
            (xe0, xo0, (i00, i01), r0, gsem0, wsem0),
            (xe1, xo1, (i10, i11), r1, gsem1, wsem1),
        ]

        rows0 = sid * (NROWP // 16)
        pltpu.sync_copy(tp_hbm.at[pl.ds(rows0, NROWP // 16)],
                        tp_sp.at[pl.ds(rows0, NROWP // 16)])
        plsc.subcore_barrier()

        def prep(g, slot):
            xe_b, xo_b, ibs, rows_b, gsem, _ = slot
            base = c0 + wid * ROWS_W + g * BLK
            pltpu.sync_copy(xe_hbm.at[pl.ds(base, BLK)], xe_b)
            pltpu.sync_copy(xo_hbm.at[pl.ds(base, BLK)], xo_b)
            for j in range(BLK // 16):
                xe = xe_b[pl.ds(j * 16, 16)]
                xo = xo_b[pl.ds(j * 16, 16)]
                p = lax.rem(base + j * 16 + lane, NP)
                ibs[j // 8][pl.ds((j % 8) * 16, 16)] = (
                    p * (NSEEN * NSEEN) + xe * NSEEN + xo)
            return [
                pltpu.async_copy(tp_sp.at[ibs[s]],
                                 rows_b.at[pl.ds(s * 128, 128)], gsem)
                for s in range(NSTREAM)
            ]

        pend_g = {0: prep(0, slots[0]), 1: None}
        pend_w = {0: None, 1: None}
        for g in range(NBLK):
            s = g % 2
            s2 = (g + 1) % 2
            if g + 1 < NBLK:
                if pend_w[s2] is not None:
                    pend_w[s2].wait()
                pend_g[s2] = prep(g + 1, slots[s2])
            for c in pend_g[s]:
                c.wait()
            loc = wid * ROWS_W + g * BLK
            pend_w[s] = pltpu.async_copy(
                slots[s][3], out_hbm.at[pl.ds(loc, BLK)], slots[s][5])
        pend_w[(NBLK - 1) % 2].wait()
        pend_w[NBLK % 2].wait()

    return _gather_body


def kernel(X, emb_w, def_w, w_w):
    emb_e = emb_w[:, :NSEEN, :].reshape(FN, D)
    w_e = w_w[:, :NSEEN, :].reshape(FN, 1)
    def_e = jnp.broadcast_to(def_w[:, None, :], (F, NSEEN, D)).reshape(FN, D)
    t = pl.pallas_call(
        _fuse_body,
        out_shape=jax.ShapeDtypeStruct((FN, D), jnp.float32),
    )(emb_e, w_e, def_e)

    t3 = t.reshape(NP, 2, NSEEN, D)
    te = jnp.broadcast_to(t3[:, 0, :, None, :], (NP, NSEEN, NSEEN, D))
    to = jnp.broadcast_to(t3[:, 1, None, :, :], (NP, NSEEN, NSEEN, D))
    tp = jnp.concatenate([te, to], axis=-1).reshape(NROW, 2 * D)
    tp = jnp.pad(tp, ((0, NROWP - NROW), (0, 0)))

    xe_flat = X[:, 0::2].reshape(BP)
    xo_flat = X[:, 1::2].reshape(BP)

    mesh = plsc.VectorSubcoreMesh(core_axis_name="c", subcore_axis_name="s")
    scratch = [
        pltpu.VMEM((BLK,), jnp.int32),
        pltpu.VMEM((BLK,), jnp.int32),
        pltpu.VMEM((BLK,), jnp.int32),
        pltpu.VMEM((BLK,), jnp.int32),
        pltpu.VMEM((128,), jnp.int32),
        pltpu.VMEM((128,), jnp.int32),
        pltpu.VMEM((128,), jnp.int32),
        pltpu.VMEM((128,), jnp.int32),
        pltpu.VMEM((BLK, 2 * D), jnp.float32),
        pltpu.VMEM((BLK, 2 * D), jnp.float32),
        pltpu.VMEM_SHARED((NROWP, 2 * D), jnp.float32),
        pltpu.SemaphoreType.DMA,
        pltpu.SemaphoreType.DMA,
        pltpu.SemaphoreType.DMA,
        pltpu.SemaphoreType.DMA,
    ]

    nblk_tr = CBATCH // CB
    buf = None
    for c in range(NCHUNK):
        gather = pl.kernel(
            _make_gather_body(c),
            mesh=mesh,
            out_type=jax.ShapeDtypeStruct((CBP, 2 * D), jnp.float32),
            scratch_types=scratch,
        )
        pc = gather(xe_flat, xo_flat, tp).reshape(CBATCH, NP, 2 * D)
        if buf is None:
            buf = pl.pallas_call(
                _tr_body,
                grid=(nblk_tr,),
                in_specs=[pl.BlockSpec((CB, NP, 2 * D), lambda g: (g, 0, 0))],
                out_specs=pl.BlockSpec((NP, 2 * D, CB), lambda g: (0, 0, g)),
                out_shape=jax.ShapeDtypeStruct((NP, 2 * D, B), jnp.float32),
            )(pc)
        else:
            buf = pl.pallas_call(
                _tr_body_alias,
                grid=(nblk_tr,),
                in_specs=[
                    pl.BlockSpec(memory_space=pltpu.MemorySpace.HBM),
                    pl.BlockSpec((CB, NP, 2 * D), lambda g: (g, 0, 0)),
                ],
                out_specs=pl.BlockSpec(
                    (NP, 2 * D, CB),
                    lambda g, cc=c: (0, 0, cc * nblk_tr + g)),
                out_shape=jax.ShapeDtypeStruct((NP, 2 * D, B), jnp.float32),
                input_output_aliases={0: 0},
            )(buf, pc)

    return jnp.transpose(buf.reshape(F, D, B), (2, 0, 1))
